# baseline (speedup 1.0000x reference)
import functools

import jax
import jax.numpy as jnp
from jax import lax
from jax.experimental import pallas as pl
from jax.experimental.pallas import tpu as pltpu
from jax.experimental.pallas import tpu_sc as plsc

NUM_ITERS = 10
ALPHA = 0.1

NC = 2
NS = 16
CHUNK = 128
GROUP = 20


def _make_scatter_kernel(n_chunks_t, pn, ch):
    rows_per_tile = pn // NS
    n_zero = rows_per_tile // CHUNK
    n_groups = n_chunks_t // GROUP
    NBUF = 5
    n_quads = GROUP // NBUF

    @functools.partial(
        pl.kernel,
        out_type=jax.ShapeDtypeStruct((NC, pn, ch), jnp.float32),
        mesh=plsc.VectorSubcoreMesh(core_axis_name="c", subcore_axis_name="s",
                                    num_cores=NC),
        compiler_params=pltpu.CompilerParams(use_tc_tiling_on_sc=False),
        scratch_types=[
            pltpu.VMEM((GROUP, CHUNK), jnp.int32),
            pltpu.VMEM((GROUP, CHUNK), jnp.int32),
            [pltpu.VMEM((CHUNK, ch), jnp.float32)] * NBUF,
            pltpu.VMEM_SHARED((pn, ch), jnp.float32),
            [pltpu.SemaphoreType.DMA] * NBUF,
            [pltpu.SemaphoreType.DMA] * NBUF,
        ],
    )
    def scatter_kernel(src2_hbm, dst_hbm, y2_hbm, zeros_hbm, out_hbm,
                       src_idx, dst_idx, bufs, acc, semg, sems):
        cid = lax.axis_index("c")
        sid = lax.axis_index("s")
        base = sid * n_chunks_t

        pltpu.sync_copy(zeros_hbm, bufs[0])

        def zbody(r, carry):
            pltpu.sync_copy(
                bufs[0], acc.at[pl.ds(sid * rows_per_tile + r * CHUNK, CHUNK)])
            return carry

        lax.fori_loop(0, n_zero, zbody, 0, unroll=False)
        plsc.subcore_barrier()

        def gather(j, t):
            return pltpu.async_copy(y2_hbm.at[src_idx.at[j]], bufs[t], semg[t])

        def wait_gather(j, t):
            pltpu.make_async_copy(y2_hbm.at[src_idx.at[j]], bufs[t],
                                  semg[t]).wait()

        def scatter(j, t):
            return pltpu.async_copy(bufs[t], acc.at[dst_idx.at[j]], sems[t],
                                    add=True)

        def wait_scatter(j, t):
            pltpu.make_async_copy(bufs[t], acc.at[dst_idx.at[j]],
                                  sems[t]).wait()

        def group(g, carry):
            pltpu.sync_copy(
                src2_hbm.at[cid, pl.ds(base + g * GROUP, GROUP)], src_idx)
            pltpu.sync_copy(
                dst_hbm.at[pl.ds(base + g * GROUP, GROUP)], dst_idx)
            gather(0, 0)
            gather(1, 1)

            def quad(q, carry2):
                for t in range(NBUF):
                    j = NBUF * q + t
                    u = (t + 2) % NBUF
                    wait_gather(j, t)
                    scatter(j, t)

                    if t < 3:
                        @pl.when(q > 0)
                        def _(jj=j, uu=u):
                            wait_scatter(jj - 3, uu)

                        gather(j + 2, u)
                    else:
                        wait_scatter(j - 3, u)

                        @pl.when(q < n_quads - 1)
                        def _(jj=j, uu=u):
                            gather(jj + 2, uu)

                return carry2

            lax.fori_loop(0, n_quads, quad, 0, unroll=False)
            wait_scatter(GROUP - 3, (GROUP - 3) % NBUF)
            wait_scatter(GROUP - 2, (GROUP - 2) % NBUF)
            wait_scatter(GROUP - 1, (GROUP - 1) % NBUF)
            return carry

        lax.fori_loop(0, n_groups, group, 0, unroll=False)
        plsc.subcore_barrier()

        def obody(r, carry):
            off = sid * rows_per_tile + r * CHUNK
            pltpu.sync_copy(acc.at[pl.ds(off, CHUNK)],
                            out_hbm.at[cid, pl.ds(off, CHUNK)])
            return carry

        lax.fori_loop(0, n_zero, obody, 0, unroll=False)

    return scatter_kernel


def _prep_body(w_ref, degp_ref, p_ref, norm_ref):
    w = w_ref[...]
    m = jnp.max(w, axis=1, keepdims=True)
    e = jnp.exp(w - m)
    p_ref[...] = e / jnp.sum(e, axis=1, keepdims=True)
    nrm = 1.0 / jnp.maximum(degp_ref[0], 1.0)
    norm_ref[...] = jnp.concatenate([nrm, nrm], axis=1)


def _y0_body(est_ref, p_ref, y_ref):
    y_ref[...] = jnp.dot(est_ref[...], p_ref[...],
                         preferred_element_type=jnp.float32)


def _blend_body(parts_ref, norm_ref, est0_ref, p_ref, blend_ref, y_ref):
    s = jnp.concatenate([parts_ref[0], parts_ref[1]], axis=1)
    b = (1.0 - ALPHA) * s * norm_ref[...] + ALPHA * est0_ref[...]
    blend_ref[...] = b
    y_ref[...] = jnp.dot(b, p_ref[...], preferred_element_type=jnp.float32)


def kernel(edge_index, estimates, W):
    n, c = estimates.shape
    ch = c // 2
    e = edge_index.shape[1]

    n_chunks_t = -(-(-(-e // (NS * CHUNK))) // GROUP) * GROUP
    e_pad = n_chunks_t * CHUNK * NS
    pn = -(-n // (NS * CHUNK)) * (NS * CHUNK)
    pad = e_pad - e
    pad_ar = jnp.arange(pad, dtype=jnp.int32)
    src = jnp.concatenate([edge_index[0], pad_ar % n]).reshape(-1, CHUNK)
    dst = jnp.concatenate([edge_index[1], n + pad_ar % (pn - n)]).reshape(-1, CHUNK)
    src2 = jnp.stack([2 * src, 2 * src + 1])

    zeros_h = jnp.zeros((CHUNK, ch), jnp.float32)
    ones_t = jnp.ones((2 * n, ch), jnp.float32)

    scatter = _make_scatter_kernel(n_chunks_t, pn, ch)

    deg_parts = scatter(src2, dst, ones_t, zeros_h)

    p_mat, norm = pl.pallas_call(
        _prep_body,
        out_shape=(jax.ShapeDtypeStruct((c, c), jnp.float32),
                   jax.ShapeDtypeStruct((pn, c), jnp.float32)),
    )(W, deg_parts)

    bn = 2000
    grid = n // bn
    y = pl.pallas_call(
        _y0_body,
        grid=(grid,),
        in_specs=[pl.BlockSpec((bn, c), lambda i: (i, 0)),
                  pl.BlockSpec((c, c), lambda i: (0, 0))],
        out_specs=pl.BlockSpec((bn, c), lambda i: (i, 0)),
        out_shape=jax.ShapeDtypeStruct((n, c), jnp.float32),
    )(estimates, p_mat)

    blend_call = pl.pallas_call(
        _blend_body,
        grid=(grid,),
        in_specs=[pl.BlockSpec((NC, bn, ch), lambda i: (0, i, 0)),
                  pl.BlockSpec((bn, c), lambda i: (i, 0)),
                  pl.BlockSpec((bn, c), lambda i: (i, 0)),
                  pl.BlockSpec((c, c), lambda i: (0, 0))],
        out_specs=(pl.BlockSpec((bn, c), lambda i: (i, 0)),
                   pl.BlockSpec((bn, c), lambda i: (i, 0))),
        out_shape=(jax.ShapeDtypeStruct((n, c), jnp.float32),
                   jax.ShapeDtypeStruct((n, c), jnp.float32)),
    )

    est = estimates
    for _ in range(NUM_ITERS):
        parts = scatter(src2, dst, y.reshape(2 * n, ch), zeros_h)
        est, y = blend_call(parts, norm, estimates, p_mat)
    return est


# speedup vs baseline: 1.1868x; 1.1868x over previous
import functools

import jax
import jax.numpy as jnp
from jax import lax
from jax.experimental import pallas as pl
from jax.experimental.pallas import tpu as pltpu
from jax.experimental.pallas import tpu_sc as plsc

NUM_ITERS = 10
ALPHA = 0.1

NC = 2
NS = 16
CHUNK = 128
GROUP = 20


def _make_scatter_kernel(n_chunks_t, pn, ch):
    rows_per_tile = pn // NS
    n_zero = rows_per_tile // CHUNK
    n_groups = n_chunks_t // GROUP
    NBUF = 5
    n_quads = GROUP // NBUF

    @functools.partial(
        pl.kernel,
        out_type=jax.ShapeDtypeStruct((NC, pn, ch), jnp.float32),
        mesh=plsc.VectorSubcoreMesh(core_axis_name="c", subcore_axis_name="s",
                                    num_cores=NC),
        compiler_params=pltpu.CompilerParams(use_tc_tiling_on_sc=False),
        scratch_types=[
            pltpu.VMEM((GROUP, CHUNK), jnp.int32),
            pltpu.VMEM((GROUP, CHUNK), jnp.int32),
            [pltpu.VMEM((CHUNK, ch), jnp.float32)] * NBUF,
            pltpu.VMEM_SHARED((pn, ch), jnp.float32),
            [pltpu.SemaphoreType.DMA] * NBUF,
            [pltpu.SemaphoreType.DMA] * NBUF,
        ],
    )
    def scatter_kernel(src2_hbm, dst_hbm, y2_hbm, zeros_hbm, out_hbm,
                       src_idx, dst_idx, bufs, acc, semg, sems):
        cid = lax.axis_index("c")
        sid = lax.axis_index("s")
        base = sid * n_chunks_t

        pltpu.sync_copy(zeros_hbm, bufs[0])

        def zbody(r, carry):
            pltpu.sync_copy(
                bufs[0], acc.at[pl.ds(sid * rows_per_tile + r * CHUNK, CHUNK)])
            return carry

        lax.fori_loop(0, n_zero, zbody, 0, unroll=False)
        plsc.subcore_barrier()

        def gather(j, t):
            return pltpu.async_copy(y2_hbm.at[src_idx.at[j]], bufs[t], semg[t])

        def wait_gather(j, t):
            pltpu.make_async_copy(y2_hbm.at[src_idx.at[j]], bufs[t],
                                  semg[t]).wait()

        def scatter(j, t):
            return pltpu.async_copy(bufs[t], acc.at[dst_idx.at[j]], sems[t],
                                    add=True)

        def wait_scatter(j, t):
            pltpu.make_async_copy(bufs[t], acc.at[dst_idx.at[j]],
                                  sems[t]).wait()

        def group(g, carry):
            pltpu.sync_copy(
                src2_hbm.at[cid, pl.ds(base + g * GROUP, GROUP)], src_idx)
            pltpu.sync_copy(
                dst_hbm.at[pl.ds(base + g * GROUP, GROUP)], dst_idx)
            gather(0, 0)
            gather(1, 1)
            gather(2, 2)

            def quad(q, carry2):
                for t in range(NBUF):
                    j = NBUF * q + t
                    u = (t + 3) % NBUF
                    wait_gather(j, t)
                    scatter(j, t)

                    if t < 2:
                        @pl.when(q > 0)
                        def _(jj=j, uu=u):
                            wait_scatter(jj - 2, uu)

                        gather(j + 3, u)
                    else:
                        wait_scatter(j - 2, u)

                        @pl.when(q < n_quads - 1)
                        def _(jj=j, uu=u):
                            gather(jj + 3, uu)

                return carry2

            lax.fori_loop(0, n_quads, quad, 0, unroll=False)
            wait_scatter(GROUP - 2, (GROUP - 2) % NBUF)
            wait_scatter(GROUP - 1, (GROUP - 1) % NBUF)
            return carry

        lax.fori_loop(0, n_groups, group, 0, unroll=False)
        plsc.subcore_barrier()

        def obody(r, carry):
            off = sid * rows_per_tile + r * CHUNK
            pltpu.sync_copy(acc.at[pl.ds(off, CHUNK)],
                            out_hbm.at[cid, pl.ds(off, CHUNK)])
            return carry

        lax.fori_loop(0, n_zero, obody, 0, unroll=False)

    return scatter_kernel


def _make_degree_kernel(n_chunks_t, pn, w):
    rows_per_tile = pn // NS
    n_zero = rows_per_tile // CHUNK
    n_groups = n_chunks_t // GROUP
    NBUF = 4

    @functools.partial(
        pl.kernel,
        out_type=jax.ShapeDtypeStruct((NC, pn, w), jnp.float32),
        mesh=plsc.VectorSubcoreMesh(core_axis_name="c", subcore_axis_name="s",
                                    num_cores=NC),
        compiler_params=pltpu.CompilerParams(use_tc_tiling_on_sc=False),
        scratch_types=[
            pltpu.VMEM((GROUP, CHUNK), jnp.int32),
            pltpu.VMEM((CHUNK, w), jnp.float32),
            pltpu.VMEM((CHUNK, w), jnp.float32),
            pltpu.VMEM_SHARED((pn, w), jnp.float32),
            [pltpu.SemaphoreType.DMA] * NBUF,
        ],
    )
    def degree_kernel(dst_hbm, ones_hbm, zeros_hbm, out_hbm,
                      dst_idx, onesbuf, zbuf, acc, sems):
        cid = lax.axis_index("c")
        sid = lax.axis_index("s")
        base = sid * n_chunks_t

        pltpu.sync_copy(ones_hbm, onesbuf)
        pltpu.sync_copy(zeros_hbm, zbuf)

        def zbody(r, carry):
            pltpu.sync_copy(
                zbuf, acc.at[pl.ds(sid * rows_per_tile + r * CHUNK, CHUNK)])
            return carry

        lax.fori_loop(0, n_zero, zbody, 0, unroll=False)
        plsc.subcore_barrier()

        def scatter(j, t):
            return pltpu.async_copy(onesbuf, acc.at[dst_idx.at[j]], sems[t],
                                    add=True)

        def wait_scatter(t):
            pltpu.make_async_copy(onesbuf, acc.at[dst_idx.at[t]],
                                  sems[t]).wait()

        def group(g, carry):
            pltpu.sync_copy(
                dst_hbm.at[pl.ds(base + g * GROUP, GROUP)], dst_idx)

            def quad(q, carry2):
                for t in range(NBUF):
                    j = NBUF * q + t

                    @pl.when(q > 0)
                    def _(tt=t):
                        wait_scatter(tt)

                    scatter(j, t)

                return carry2

            lax.fori_loop(0, GROUP // NBUF, quad, 0, unroll=False)
            for t in range(NBUF):
                wait_scatter(t)
            return carry

        lax.fori_loop(0, n_groups, group, 0, unroll=False)
        plsc.subcore_barrier()

        def obody(r, carry):
            off = sid * rows_per_tile + r * CHUNK
            pltpu.sync_copy(acc.at[pl.ds(off, CHUNK)],
                            out_hbm.at[cid, pl.ds(off, CHUNK)])
            return carry

        lax.fori_loop(0, n_zero, obody, 0, unroll=False)

    return degree_kernel


def _prep_body(w_ref, degp_ref, p_ref, norm_ref):
    w = w_ref[...]
    m = jnp.max(w, axis=1, keepdims=True)
    e = jnp.exp(w - m)
    p_ref[...] = e / jnp.sum(e, axis=1, keepdims=True)
    nrm = 1.0 / jnp.maximum(degp_ref[0], 1.0)
    reps = norm_ref.shape[1] // nrm.shape[1]
    norm_ref[...] = jnp.concatenate([nrm] * reps, axis=1)


def _y0_body(est_ref, p_ref, y_ref):
    y_ref[...] = jnp.dot(est_ref[...], p_ref[...],
                         preferred_element_type=jnp.float32)


def _blend_body(parts_ref, norm_ref, est0_ref, p_ref, blend_ref, y_ref):
    s = jnp.concatenate([parts_ref[0], parts_ref[1]], axis=1)
    b = (1.0 - ALPHA) * s * norm_ref[...] + ALPHA * est0_ref[...]
    blend_ref[...] = b
    y_ref[...] = jnp.dot(b, p_ref[...], preferred_element_type=jnp.float32)


def kernel(edge_index, estimates, W):
    n, c = estimates.shape
    ch = c // 2
    e = edge_index.shape[1]

    n_chunks_t = -(-(-(-e // (NS * CHUNK))) // GROUP) * GROUP
    e_pad = n_chunks_t * CHUNK * NS
    pn = -(-n // (NS * CHUNK)) * (NS * CHUNK)
    pad = e_pad - e
    pad_ar = jnp.arange(pad, dtype=jnp.int32)
    src = jnp.concatenate([edge_index[0], pad_ar % n]).reshape(-1, CHUNK)
    dst = jnp.concatenate([edge_index[1], n + pad_ar % (pn - n)]).reshape(-1, CHUNK)
    src2 = jnp.stack([2 * src, 2 * src + 1])

    zeros_h = jnp.zeros((CHUNK, ch), jnp.float32)
    wdeg = 16
    ones_w = jnp.ones((CHUNK, wdeg), jnp.float32)
    zeros_w = jnp.zeros((CHUNK, wdeg), jnp.float32)

    scatter = _make_scatter_kernel(n_chunks_t, pn, ch)

    deg_parts = _make_degree_kernel(n_chunks_t, pn, wdeg)(dst, ones_w, zeros_w)

    p_mat, norm = pl.pallas_call(
        _prep_body,
        out_shape=(jax.ShapeDtypeStruct((c, c), jnp.float32),
                   jax.ShapeDtypeStruct((pn, c), jnp.float32)),
    )(W, deg_parts)

    bn = 2000
    grid = n // bn
    y = pl.pallas_call(
        _y0_body,
        grid=(grid,),
        in_specs=[pl.BlockSpec((bn, c), lambda i: (i, 0)),
                  pl.BlockSpec((c, c), lambda i: (0, 0))],
        out_specs=pl.BlockSpec((bn, c), lambda i: (i, 0)),
        out_shape=jax.ShapeDtypeStruct((n, c), jnp.float32),
    )(estimates, p_mat)

    blend_call = pl.pallas_call(
        _blend_body,
        grid=(grid,),
        in_specs=[pl.BlockSpec((NC, bn, ch), lambda i: (0, i, 0)),
                  pl.BlockSpec((bn, c), lambda i: (i, 0)),
                  pl.BlockSpec((bn, c), lambda i: (i, 0)),
                  pl.BlockSpec((c, c), lambda i: (0, 0))],
        out_specs=(pl.BlockSpec((bn, c), lambda i: (i, 0)),
                   pl.BlockSpec((bn, c), lambda i: (i, 0))),
        out_shape=(jax.ShapeDtypeStruct((n, c), jnp.float32),
                   jax.ShapeDtypeStruct((n, c), jnp.float32)),
    )

    est = estimates
    for _ in range(NUM_ITERS):
        parts = scatter(src2, dst, y.reshape(2 * n, ch), zeros_h)
        est, y = blend_call(parts, norm, estimates, p_mat)
    return est


# speedup vs baseline: 1.2336x; 1.0395x over previous
import functools

import jax
import jax.numpy as jnp
from jax import lax
from jax.experimental import pallas as pl
from jax.experimental.pallas import tpu as pltpu
from jax.experimental.pallas import tpu_sc as plsc

NUM_ITERS = 10
ALPHA = 0.1

NC = 2
NS = 16
CHUNK = 128
GROUP = 20


def _make_scatter_kernel(n_chunks_t, pn, ch):
    rows_per_tile = pn // NS
    n_zero = rows_per_tile // CHUNK
    n_groups = n_chunks_t // GROUP
    NBUF = 5
    n_quads = GROUP // NBUF

    @functools.partial(
        pl.kernel,
        out_type=jax.ShapeDtypeStruct((NC, pn, ch), jnp.float32),
        mesh=plsc.VectorSubcoreMesh(core_axis_name="c", subcore_axis_name="s",
                                    num_cores=NC),
        compiler_params=pltpu.CompilerParams(use_tc_tiling_on_sc=False),
        scratch_types=[
            pltpu.VMEM((GROUP, CHUNK), jnp.int32),
            pltpu.VMEM((GROUP, CHUNK), jnp.int32),
            [pltpu.VMEM((CHUNK, ch), jnp.float32)] * NBUF,
            pltpu.VMEM_SHARED((pn, ch), jnp.float32),
            [pltpu.SemaphoreType.DMA] * NBUF,
            [pltpu.SemaphoreType.DMA] * NBUF,
        ],
    )
    def scatter_kernel(src2_hbm, dst_hbm, y2_hbm, zeros_hbm, out_hbm,
                       src_idx, dst_idx, bufs, acc, semg, sems):
        cid = lax.axis_index("c")
        sid = lax.axis_index("s")
        base = sid * n_chunks_t

        pltpu.sync_copy(zeros_hbm, bufs[0])

        def zfire(r, carry):
            pltpu.async_copy(
                bufs[0], acc.at[pl.ds(sid * rows_per_tile + r * CHUNK, CHUNK)],
                semg[0])
            return carry

        def zdrain(r, carry):
            pltpu.make_async_copy(
                bufs[0], acc.at[pl.ds(sid * rows_per_tile + r * CHUNK, CHUNK)],
                semg[0]).wait()
            return carry

        lax.fori_loop(0, n_zero, zfire, 0, unroll=False)
        lax.fori_loop(0, n_zero, zdrain, 0, unroll=False)
        plsc.subcore_barrier()

        def gather(j, t):
            return pltpu.async_copy(y2_hbm.at[src_idx.at[j]], bufs[t], semg[t])

        def wait_gather(j, t):
            pltpu.make_async_copy(y2_hbm.at[src_idx.at[j]], bufs[t],
                                  semg[t]).wait()

        def scatter(j, t):
            return pltpu.async_copy(bufs[t], acc.at[dst_idx.at[j]], sems[t],
                                    add=True)

        def wait_scatter(j, t):
            pltpu.make_async_copy(bufs[t], acc.at[dst_idx.at[j]],
                                  sems[t]).wait()

        def group(g, carry):
            cp_s = pltpu.async_copy(
                src2_hbm.at[cid, pl.ds(base + g * GROUP, GROUP)], src_idx,
                semg[0])
            cp_d = pltpu.async_copy(
                dst_hbm.at[pl.ds(base + g * GROUP, GROUP)], dst_idx, semg[1])
            cp_s.wait()
            cp_d.wait()
            gather(0, 0)
            gather(1, 1)
            gather(2, 2)

            def quad(q, carry2):
                for t in range(NBUF):
                    j = NBUF * q + t
                    u = (t + 3) % NBUF
                    wait_gather(j, t)
                    scatter(j, t)

                    if t < 2:
                        @pl.when(q > 0)
                        def _(jj=j, uu=u):
                            wait_scatter(jj - 2, uu)

                        gather(j + 3, u)
                    else:
                        wait_scatter(j - 2, u)

                        @pl.when(q < n_quads - 1)
                        def _(jj=j, uu=u):
                            gather(jj + 3, uu)

                return carry2

            lax.fori_loop(0, n_quads, quad, 0, unroll=False)
            wait_scatter(GROUP - 2, (GROUP - 2) % NBUF)
            wait_scatter(GROUP - 1, (GROUP - 1) % NBUF)
            return carry

        lax.fori_loop(0, n_groups, group, 0, unroll=False)
        plsc.subcore_barrier()

        def ofire(r, carry):
            off = sid * rows_per_tile + r * CHUNK
            pltpu.async_copy(acc.at[pl.ds(off, CHUNK)],
                             out_hbm.at[cid, pl.ds(off, CHUNK)], semg[0])
            return carry

        def odrain(r, carry):
            off = sid * rows_per_tile + r * CHUNK
            pltpu.make_async_copy(acc.at[pl.ds(off, CHUNK)],
                                  out_hbm.at[cid, pl.ds(off, CHUNK)],
                                  semg[0]).wait()
            return carry

        lax.fori_loop(0, n_zero, ofire, 0, unroll=False)
        lax.fori_loop(0, n_zero, odrain, 0, unroll=False)

    return scatter_kernel


def _make_degree_kernel(n_chunks_t, pn, w):
    rows_per_tile = pn // NS
    n_zero = rows_per_tile // CHUNK
    n_groups = n_chunks_t // GROUP
    NBUF = 4

    @functools.partial(
        pl.kernel,
        out_type=jax.ShapeDtypeStruct((NC, pn, w), jnp.float32),
        mesh=plsc.VectorSubcoreMesh(core_axis_name="c", subcore_axis_name="s",
                                    num_cores=NC),
        compiler_params=pltpu.CompilerParams(use_tc_tiling_on_sc=False),
        scratch_types=[
            pltpu.VMEM((GROUP, CHUNK), jnp.int32),
            pltpu.VMEM((CHUNK, w), jnp.float32),
            pltpu.VMEM((CHUNK, w), jnp.float32),
            pltpu.VMEM_SHARED((pn, w), jnp.float32),
            [pltpu.SemaphoreType.DMA] * NBUF,
        ],
    )
    def degree_kernel(dst_hbm, ones_hbm, zeros_hbm, out_hbm,
                      dst_idx, onesbuf, zbuf, acc, sems):
        cid = lax.axis_index("c")
        sid = lax.axis_index("s")
        base = sid * n_chunks_t

        pltpu.sync_copy(ones_hbm, onesbuf)
        pltpu.sync_copy(zeros_hbm, zbuf)

        def zbody(r, carry):
            pltpu.sync_copy(
                zbuf, acc.at[pl.ds(sid * rows_per_tile + r * CHUNK, CHUNK)])
            return carry

        lax.fori_loop(0, n_zero, zbody, 0, unroll=False)
        plsc.subcore_barrier()

        def scatter(j, t):
            return pltpu.async_copy(onesbuf, acc.at[dst_idx.at[j]], sems[t],
                                    add=True)

        def wait_scatter(t):
            pltpu.make_async_copy(onesbuf, acc.at[dst_idx.at[t]],
                                  sems[t]).wait()

        def group(g, carry):
            pltpu.sync_copy(
                dst_hbm.at[pl.ds(base + g * GROUP, GROUP)], dst_idx)

            def quad(q, carry2):
                for t in range(NBUF):
                    j = NBUF * q + t

                    @pl.when(q > 0)
                    def _(tt=t):
                        wait_scatter(tt)

                    scatter(j, t)

                return carry2

            lax.fori_loop(0, GROUP // NBUF, quad, 0, unroll=False)
            for t in range(NBUF):
                wait_scatter(t)
            return carry

        lax.fori_loop(0, n_groups, group, 0, unroll=False)
        plsc.subcore_barrier()

        def obody(r, carry):
            off = sid * rows_per_tile + r * CHUNK
            pltpu.sync_copy(acc.at[pl.ds(off, CHUNK)],
                            out_hbm.at[cid, pl.ds(off, CHUNK)])
            return carry

        lax.fori_loop(0, n_zero, obody, 0, unroll=False)

    return degree_kernel


def _prep_body(w_ref, degp_ref, p_ref, norm_ref):
    w = w_ref[...]
    m = jnp.max(w, axis=1, keepdims=True)
    e = jnp.exp(w - m)
    p_ref[...] = e / jnp.sum(e, axis=1, keepdims=True)
    nrm = 1.0 / jnp.maximum(degp_ref[0], 1.0)
    reps = norm_ref.shape[1] // nrm.shape[1]
    norm_ref[...] = jnp.concatenate([nrm] * reps, axis=1)


def _y0_body(est_ref, p_ref, y_ref):
    y_ref[...] = jnp.dot(est_ref[...], p_ref[...],
                         preferred_element_type=jnp.float32)


def _blend_body(parts_ref, norm_ref, est0_ref, p_ref, blend_ref, y_ref):
    s = jnp.concatenate([parts_ref[0], parts_ref[1]], axis=1)
    b = (1.0 - ALPHA) * s * norm_ref[...] + ALPHA * est0_ref[...]
    blend_ref[...] = b
    y_ref[...] = jnp.dot(b, p_ref[...], preferred_element_type=jnp.float32)


def kernel(edge_index, estimates, W):
    n, c = estimates.shape
    ch = c // 2
    e = edge_index.shape[1]

    n_chunks_t = -(-(-(-e // (NS * CHUNK))) // GROUP) * GROUP
    e_pad = n_chunks_t * CHUNK * NS
    pn = -(-n // (NS * CHUNK)) * (NS * CHUNK)
    pad = e_pad - e
    pad_ar = jnp.arange(pad, dtype=jnp.int32)
    src = jnp.concatenate([edge_index[0], pad_ar % n]).reshape(-1, CHUNK)
    dst = jnp.concatenate([edge_index[1], n + pad_ar % (pn - n)]).reshape(-1, CHUNK)
    src2 = jnp.stack([2 * src, 2 * src + 1])

    zeros_h = jnp.zeros((CHUNK, ch), jnp.float32)
    wdeg = 16
    ones_w = jnp.ones((CHUNK, wdeg), jnp.float32)
    zeros_w = jnp.zeros((CHUNK, wdeg), jnp.float32)

    scatter = _make_scatter_kernel(n_chunks_t, pn, ch)

    deg_parts = _make_degree_kernel(n_chunks_t, pn, wdeg)(dst, ones_w, zeros_w)

    p_mat, norm = pl.pallas_call(
        _prep_body,
        out_shape=(jax.ShapeDtypeStruct((c, c), jnp.float32),
                   jax.ShapeDtypeStruct((pn, c), jnp.float32)),
    )(W, deg_parts)

    bn = 2000
    grid = n // bn
    y = pl.pallas_call(
        _y0_body,
        grid=(grid,),
        in_specs=[pl.BlockSpec((bn, c), lambda i: (i, 0)),
                  pl.BlockSpec((c, c), lambda i: (0, 0))],
        out_specs=pl.BlockSpec((bn, c), lambda i: (i, 0)),
        out_shape=jax.ShapeDtypeStruct((n, c), jnp.float32),
    )(estimates, p_mat)

    blend_call = pl.pallas_call(
        _blend_body,
        grid=(grid,),
        in_specs=[pl.BlockSpec((NC, bn, ch), lambda i: (0, i, 0)),
                  pl.BlockSpec((bn, c), lambda i: (i, 0)),
                  pl.BlockSpec((bn, c), lambda i: (i, 0)),
                  pl.BlockSpec((c, c), lambda i: (0, 0))],
        out_specs=(pl.BlockSpec((bn, c), lambda i: (i, 0)),
                   pl.BlockSpec((bn, c), lambda i: (i, 0))),
        out_shape=(jax.ShapeDtypeStruct((n, c), jnp.float32),
                   jax.ShapeDtypeStruct((n, c), jnp.float32)),
    )

    est = estimates
    for _ in range(NUM_ITERS):
        parts = scatter(src2, dst, y.reshape(2 * n, ch), zeros_h)
        est, y = blend_call(parts, norm, estimates, p_mat)
    return est


# speedup vs baseline: 1.2514x; 1.0145x over previous
import functools

import jax
import jax.numpy as jnp
from jax import lax
from jax.experimental import pallas as pl
from jax.experimental.pallas import tpu as pltpu
from jax.experimental.pallas import tpu_sc as plsc

NUM_ITERS = 10
ALPHA = 0.1

NC = 2
NS = 16
CHUNK = 128
GROUP = 20


def _make_scatter_kernel(n_chunks_t, pn, ch):
    rows_per_tile = pn // NS
    n_zero = rows_per_tile // CHUNK
    n_groups = n_chunks_t // GROUP
    NBUF = 5
    n_quads = GROUP // NBUF

    @functools.partial(
        pl.kernel,
        out_type=jax.ShapeDtypeStruct((NC, pn, ch), jnp.float32),
        mesh=plsc.VectorSubcoreMesh(core_axis_name="c", subcore_axis_name="s",
                                    num_cores=NC),
        compiler_params=pltpu.CompilerParams(use_tc_tiling_on_sc=False),
        scratch_types=[
            pltpu.VMEM((GROUP, CHUNK), jnp.int32),
            pltpu.VMEM((GROUP, CHUNK), jnp.int32),
            [pltpu.VMEM((CHUNK, ch), jnp.float32)] * NBUF,
            pltpu.VMEM_SHARED((pn, ch), jnp.float32),
            [pltpu.SemaphoreType.DMA] * NBUF,
            [pltpu.SemaphoreType.DMA] * NBUF,
        ],
    )
    def scatter_kernel(src2_hbm, dst_hbm, y2_hbm, zeros_hbm, out_hbm,
                       src_idx, dst_idx, bufs, acc, semg, sems):
        cid = lax.axis_index("c")
        sid = lax.axis_index("s")
        base = sid * n_chunks_t

        pltpu.sync_copy(zeros_hbm, bufs[0])

        def zfire(r, carry):
            pltpu.async_copy(
                bufs[0], acc.at[pl.ds(sid * rows_per_tile + r * CHUNK, CHUNK)],
                semg[0])
            return carry

        def zdrain(r, carry):
            pltpu.make_async_copy(
                bufs[0], acc.at[pl.ds(sid * rows_per_tile + r * CHUNK, CHUNK)],
                semg[0]).wait()
            return carry

        lax.fori_loop(0, n_zero, zfire, 0, unroll=False)
        lax.fori_loop(0, n_zero, zdrain, 0, unroll=False)
        plsc.subcore_barrier()

        def gather(j, t):
            return pltpu.async_copy(y2_hbm.at[src_idx.at[j]], bufs[t], semg[t])

        def wait_gather(j, t):
            pltpu.make_async_copy(y2_hbm.at[src_idx.at[j]], bufs[t],
                                  semg[t]).wait()

        def scatter(j, t):
            return pltpu.async_copy(bufs[t], acc.at[dst_idx.at[j]], sems[t],
                                    add=True)

        def wait_scatter(j, t):
            pltpu.make_async_copy(bufs[t], acc.at[dst_idx.at[j]],
                                  sems[t]).wait()

        def group(g, carry):
            cp_s = pltpu.async_copy(
                src2_hbm.at[cid, pl.ds(base + g * GROUP, GROUP)], src_idx,
                semg[0])
            cp_d = pltpu.async_copy(
                dst_hbm.at[pl.ds(base + g * GROUP, GROUP)], dst_idx, semg[1])
            cp_s.wait()
            cp_d.wait()
            gather(0, 0)
            gather(1, 1)
            gather(2, 2)

            def quad(q, carry2):
                for t in range(NBUF):
                    j = NBUF * q + t
                    u = (t + 3) % NBUF
                    wait_gather(j, t)
                    scatter(j, t)

                    if t < 2:
                        @pl.when(q > 0)
                        def _(jj=j, uu=u):
                            wait_scatter(jj - 2, uu)

                        gather(j + 3, u)
                    else:
                        wait_scatter(j - 2, u)

                        @pl.when(q < n_quads - 1)
                        def _(jj=j, uu=u):
                            gather(jj + 3, uu)

                return carry2

            lax.fori_loop(0, n_quads, quad, 0, unroll=False)
            wait_scatter(GROUP - 2, (GROUP - 2) % NBUF)
            wait_scatter(GROUP - 1, (GROUP - 1) % NBUF)
            return carry

        lax.fori_loop(0, n_groups, group, 0, unroll=False)
        plsc.subcore_barrier()

        def ofire(r, carry):
            off = sid * rows_per_tile + r * CHUNK
            pltpu.async_copy(acc.at[pl.ds(off, CHUNK)],
                             out_hbm.at[cid, pl.ds(off, CHUNK)], semg[0])
            return carry

        def odrain(r, carry):
            off = sid * rows_per_tile + r * CHUNK
            pltpu.make_async_copy(acc.at[pl.ds(off, CHUNK)],
                                  out_hbm.at[cid, pl.ds(off, CHUNK)],
                                  semg[0]).wait()
            return carry

        lax.fori_loop(0, n_zero, ofire, 0, unroll=False)
        lax.fori_loop(0, n_zero, odrain, 0, unroll=False)

    return scatter_kernel


def _make_degree_kernel(n_chunks_t, pn, w):
    rows_per_tile = pn // NS
    n_zero = rows_per_tile // CHUNK
    n_groups = n_chunks_t // GROUP
    NBUF = 4

    @functools.partial(
        pl.kernel,
        out_type=jax.ShapeDtypeStruct((NC, pn, w), jnp.float32),
        mesh=plsc.VectorSubcoreMesh(core_axis_name="c", subcore_axis_name="s",
                                    num_cores=NC),
        compiler_params=pltpu.CompilerParams(use_tc_tiling_on_sc=False),
        scratch_types=[
            pltpu.VMEM((GROUP, CHUNK), jnp.int32),
            pltpu.VMEM((CHUNK, w), jnp.float32),
            pltpu.VMEM((CHUNK, w), jnp.float32),
            pltpu.VMEM_SHARED((pn, w), jnp.float32),
            [pltpu.SemaphoreType.DMA] * NBUF,
        ],
    )
    def degree_kernel(dst_hbm, ones_hbm, zeros_hbm, out_hbm,
                      dst_idx, onesbuf, zbuf, acc, sems):
        cid = lax.axis_index("c")
        sid = lax.axis_index("s")
        base = sid * n_chunks_t

        pltpu.sync_copy(ones_hbm, onesbuf)
        pltpu.sync_copy(zeros_hbm, zbuf)

        def zbody(r, carry):
            pltpu.sync_copy(
                zbuf, acc.at[pl.ds(sid * rows_per_tile + r * CHUNK, CHUNK)])
            return carry

        lax.fori_loop(0, n_zero, zbody, 0, unroll=False)
        plsc.subcore_barrier()

        def scatter(j, t):
            return pltpu.async_copy(onesbuf, acc.at[dst_idx.at[j]], sems[t],
                                    add=True)

        def wait_scatter(t):
            pltpu.make_async_copy(onesbuf, acc.at[dst_idx.at[t]],
                                  sems[t]).wait()

        def group(g, carry):
            pltpu.sync_copy(
                dst_hbm.at[pl.ds(base + g * GROUP, GROUP)], dst_idx)

            def quad(q, carry2):
                for t in range(NBUF):
                    j = NBUF * q + t

                    @pl.when(q > 0)
                    def _(tt=t):
                        wait_scatter(tt)

                    scatter(j, t)

                return carry2

            lax.fori_loop(0, GROUP // NBUF, quad, 0, unroll=False)
            for t in range(NBUF):
                wait_scatter(t)
            return carry

        lax.fori_loop(0, n_groups, group, 0, unroll=False)
        plsc.subcore_barrier()

        def obody(r, carry):
            off = sid * rows_per_tile + r * CHUNK
            pltpu.sync_copy(acc.at[pl.ds(off, CHUNK)],
                            out_hbm.at[cid, pl.ds(off, CHUNK)])
            return carry

        lax.fori_loop(0, n_zero, obody, 0, unroll=False)

    return degree_kernel


def _prep_body(w_ref, degp_ref, est_ref, p_ref, norm_ref, y_ref):
    w = w_ref[...]
    m = jnp.max(w, axis=1, keepdims=True)
    e = jnp.exp(w - m)
    p = e / jnp.sum(e, axis=1, keepdims=True)
    p_ref[...] = p
    nrm = 1.0 / jnp.maximum(degp_ref[0], 1.0)
    reps = norm_ref.shape[1] // nrm.shape[1]
    norm_ref[...] = jnp.concatenate([nrm] * reps, axis=1)
    y_ref[...] = jnp.dot(est_ref[...], p, preferred_element_type=jnp.float32)


def _blend_body(parts_ref, norm_ref, est0_ref, p_ref, blend_ref, y_ref):
    s = jnp.concatenate([parts_ref[0], parts_ref[1]], axis=1)
    b = (1.0 - ALPHA) * s * norm_ref[...] + ALPHA * est0_ref[...]
    blend_ref[...] = b
    y_ref[...] = jnp.dot(b, p_ref[...], preferred_element_type=jnp.float32)


def kernel(edge_index, estimates, W):
    n, c = estimates.shape
    ch = c // 2
    e = edge_index.shape[1]

    n_chunks_t = -(-(-(-e // (NS * CHUNK))) // GROUP) * GROUP
    e_pad = n_chunks_t * CHUNK * NS
    pn = -(-n // (NS * CHUNK)) * (NS * CHUNK)
    pad = e_pad - e
    pad_ar = jnp.arange(pad, dtype=jnp.int32)
    src = jnp.concatenate([edge_index[0], pad_ar % n]).reshape(-1, CHUNK)
    dst = jnp.concatenate([edge_index[1], n + pad_ar % (pn - n)]).reshape(-1, CHUNK)
    src2 = jnp.stack([2 * src, 2 * src + 1])

    zeros_h = jnp.zeros((CHUNK, ch), jnp.float32)
    wdeg = 16
    ones_w = jnp.ones((CHUNK, wdeg), jnp.float32)
    zeros_w = jnp.zeros((CHUNK, wdeg), jnp.float32)

    scatter = _make_scatter_kernel(n_chunks_t, pn, ch)

    deg_parts = _make_degree_kernel(n_chunks_t, pn, wdeg)(dst, ones_w, zeros_w)

    bn = 5000
    grid = n // bn
    bpn = pn // grid
    p_mat, norm, y = pl.pallas_call(
        _prep_body,
        grid=(grid,),
        in_specs=[pl.BlockSpec((c, c), lambda i: (0, 0)),
                  pl.BlockSpec((NC, bpn, wdeg), lambda i: (0, i, 0)),
                  pl.BlockSpec((bn, c), lambda i: (i, 0))],
        out_specs=(pl.BlockSpec((c, c), lambda i: (0, 0)),
                   pl.BlockSpec((bpn, c), lambda i: (i, 0)),
                   pl.BlockSpec((bn, c), lambda i: (i, 0))),
        out_shape=(jax.ShapeDtypeStruct((c, c), jnp.float32),
                   jax.ShapeDtypeStruct((pn, c), jnp.float32),
                   jax.ShapeDtypeStruct((n, c), jnp.float32)),
    )(W, deg_parts, estimates)

    blend_call = pl.pallas_call(
        _blend_body,
        grid=(grid,),
        in_specs=[pl.BlockSpec((NC, bn, ch), lambda i: (0, i, 0)),
                  pl.BlockSpec((bn, c), lambda i: (i, 0)),
                  pl.BlockSpec((bn, c), lambda i: (i, 0)),
                  pl.BlockSpec((c, c), lambda i: (0, 0))],
        out_specs=(pl.BlockSpec((bn, c), lambda i: (i, 0)),
                   pl.BlockSpec((bn, c), lambda i: (i, 0))),
        out_shape=(jax.ShapeDtypeStruct((n, c), jnp.float32),
                   jax.ShapeDtypeStruct((n, c), jnp.float32)),
    )

    est = estimates
    for _ in range(NUM_ITERS):
        parts = scatter(src2, dst, y.reshape(2 * n, ch), zeros_h)
        est, y = blend_call(parts, norm, estimates, p_mat)
    return est
